# Initial kernel scaffold; baseline (speedup 1.0000x reference)
import functools

import jax
import jax.numpy as jnp
from jax import lax
from jax.experimental import pallas as pl
from jax.experimental.pallas import tpu as pltpu
from jax.experimental.pallas import tpu_sc as plsc

N_ATOMS = 1600000
N_MOL = 50000
NC = 2
NS = 16
L = 16
NW = NC * NS
CHUNK = N_ATOMS // NW
BLK = 2000
NBLK = CHUNK // BLK
P = 50176
PS = P // NS


def _sc_segment_sum_body(e_hbm, i_hbm, out_hbm, e_v, i_v, z_v, acc_sh, sem):
    c = lax.axis_index("c")
    s = lax.axis_index("s")
    wid = c * NS + s

    @pl.loop(0, PS, step=L)
    def _zero(j):
        z_v[pl.ds(j, L)] = jnp.zeros((L,), jnp.float32)

    pltpu.sync_copy(z_v, acc_sh.at[pl.ds(s * PS, PS)])
    plsc.subcore_barrier()

    base = wid * CHUNK

    @pl.loop(0, NBLK)
    def _block(b):
        off = base + b * BLK
        pltpu.sync_copy(e_hbm.at[pl.ds(off, BLK)], e_v)
        pltpu.sync_copy(i_hbm.at[pl.ds(off, BLK)], i_v)
        pltpu.sync_copy(e_v, acc_sh.at[i_v], add=True)

    plsc.subcore_barrier()
    pltpu.sync_copy(acc_sh.at[pl.ds(s * PS, PS)],
                    out_hbm.at[c, pl.ds(s * PS, PS)])


@jax.jit
def _sc_segment_sum(energies, ids):
    mesh = plsc.VectorSubcoreMesh(core_axis_name="c", subcore_axis_name="s")
    return pl.kernel(
        _sc_segment_sum_body,
        out_type=jax.ShapeDtypeStruct((NC, P), jnp.float32),
        mesh=mesh,
        scratch_types=[
            pltpu.VMEM((BLK,), jnp.float32),
            pltpu.VMEM((BLK,), jnp.int32),
            pltpu.VMEM((PS,), jnp.float32),
            pltpu.VMEM_SHARED((P,), jnp.float32),
            pltpu.SemaphoreType.DMA,
        ],
    )(energies, ids)


def _tc_combine_body(p_ref, o_ref):
    o_ref[...] = p_ref[0] + p_ref[1]


@jax.jit
def _tc_combine(partials):
    return pl.pallas_call(
        _tc_combine_body,
        out_shape=jax.ShapeDtypeStruct((P,), jnp.float32),
    )(partials)


def kernel(atomic_energies, batch):
    ids = batch.astype(jnp.int32)
    partials = _sc_segment_sum(atomic_energies, ids)
    return _tc_combine(partials)[:N_MOL]


# speedup vs baseline: 20.7633x; 20.7633x over previous
import functools

import jax
import jax.numpy as jnp
from jax import lax
from jax.experimental import pallas as pl
from jax.experimental.pallas import tpu as pltpu
from jax.experimental.pallas import tpu_sc as plsc

N_ATOMS = 1600000
N_MOL = 50000
NC = 2
NS = 16
L = 16
NW = NC * NS
CHUNK = N_ATOMS // NW
BLK = 2000
NBLK = CHUNK // BLK
P = 50176
PS = P // NS


def _sc_segment_sum_body(e_hbm, i_hbm, out_hbm, e_v, i_v, z_v, acc_sh, sem):
    c = lax.axis_index("c")
    s = lax.axis_index("s")
    wid = c * NS + s

    @pl.loop(0, PS, step=L)
    def _zero(j):
        z_v[pl.ds(j, L)] = jnp.zeros((L,), jnp.float32)

    pltpu.sync_copy(z_v, acc_sh.at[pl.ds(s * PS, PS)])
    plsc.subcore_barrier()

    base = wid * CHUNK

    @pl.loop(0, NBLK)
    def _block(b):
        off = base + b * BLK
        pltpu.sync_copy(e_hbm.at[pl.ds(off, BLK)], e_v)
        pltpu.sync_copy(i_hbm.at[pl.ds(off, BLK)], i_v)
        pltpu.sync_copy(e_v, acc_sh.at[i_v], add=True)

    plsc.subcore_barrier()
    pltpu.sync_copy(acc_sh.at[pl.ds(s * PS, PS)], z_v)
    pltpu.sync_copy(z_v, out_hbm.at[pl.ds(c * P + s * PS, PS)])


@jax.jit
def _sc_segment_sum(energies, ids):
    mesh = plsc.VectorSubcoreMesh(core_axis_name="c", subcore_axis_name="s")
    return pl.kernel(
        _sc_segment_sum_body,
        out_type=jax.ShapeDtypeStruct((NC * P,), jnp.float32),
        mesh=mesh,
        scratch_types=[
            pltpu.VMEM((BLK,), jnp.float32),
            pltpu.VMEM((BLK,), jnp.int32),
            pltpu.VMEM((PS,), jnp.float32),
            pltpu.VMEM_SHARED((P,), jnp.float32),
            pltpu.SemaphoreType.DMA,
        ],
    )(energies, ids)


def _tc_combine_body(p_ref, o_ref):
    o_ref[...] = p_ref[0] + p_ref[1]


@jax.jit
def _combine(partials_flat):
    return _tc_combine(partials_flat.reshape(NC, P))


@jax.jit
def _tc_combine(partials):
    return pl.pallas_call(
        _tc_combine_body,
        out_shape=jax.ShapeDtypeStruct((P,), jnp.float32),
    )(partials)


def kernel(atomic_energies, batch):
    ids = batch.astype(jnp.int32)
    partials = _sc_segment_sum(atomic_energies, ids)
    return _combine(partials)[:N_MOL]


# speedup vs baseline: 30.8201x; 1.4844x over previous
import functools

import jax
import jax.numpy as jnp
from jax import lax
from jax.experimental import pallas as pl
from jax.experimental.pallas import tpu as pltpu
from jax.experimental.pallas import tpu_sc as plsc

N_ATOMS = 1600000
N_MOL = 50000
NC = 2
NS = 16
L = 16
NW = NC * NS
CHUNK = N_ATOMS // NW
BLK = 5000
NBLK = CHUNK // BLK
P = 50176
PS = P // NS


def _sc_segment_sum_body(e_hbm, i_hbm, out_hbm,
                         e_v0, i_v0, e_v1, i_v1, z_v, acc_sh, sem0, sem1):
    c = lax.axis_index("c")
    s = lax.axis_index("s")
    wid = c * NS + s

    @pl.loop(0, PS, step=L)
    def _zero(j):
        z_v[pl.ds(j, L)] = jnp.zeros((L,), jnp.float32)

    pltpu.sync_copy(z_v, acc_sh.at[pl.ds(s * PS, PS)])
    plsc.subcore_barrier()

    base = wid * CHUNK

    def start_load(b, e_v, i_v, sem):
        off = base + b * BLK
        pltpu.make_async_copy(e_hbm.at[pl.ds(off, BLK)], e_v, sem).start()
        pltpu.make_async_copy(i_hbm.at[pl.ds(off, BLK)], i_v, sem).start()

    def wait_load(e_v, i_v, sem):
        pltpu.make_async_copy(e_hbm.at[pl.ds(base, BLK)], e_v, sem).wait()
        pltpu.make_async_copy(i_hbm.at[pl.ds(base, BLK)], i_v, sem).wait()

    start_load(0, e_v0, i_v0, sem0)

    @pl.loop(0, NBLK, step=2)
    def _block(b):
        wait_load(e_v0, i_v0, sem0)
        start_load(b + 1, e_v1, i_v1, sem1)
        pltpu.sync_copy(e_v0, acc_sh.at[i_v0], add=True)
        wait_load(e_v1, i_v1, sem1)

        @pl.when(b + 2 < NBLK)
        def _():
            start_load(b + 2, e_v0, i_v0, sem0)

        pltpu.sync_copy(e_v1, acc_sh.at[i_v1], add=True)

    plsc.subcore_barrier()
    pltpu.sync_copy(acc_sh.at[pl.ds(s * PS, PS)], z_v)
    pltpu.sync_copy(z_v, out_hbm.at[pl.ds(c * P + s * PS, PS)])


@jax.jit
def _sc_segment_sum(energies, ids):
    mesh = plsc.VectorSubcoreMesh(core_axis_name="c", subcore_axis_name="s")
    return pl.kernel(
        _sc_segment_sum_body,
        out_type=jax.ShapeDtypeStruct((NC * P,), jnp.float32),
        mesh=mesh,
        scratch_types=[
            pltpu.VMEM((BLK,), jnp.float32),
            pltpu.VMEM((BLK,), jnp.int32),
            pltpu.VMEM((BLK,), jnp.float32),
            pltpu.VMEM((BLK,), jnp.int32),
            pltpu.VMEM((PS,), jnp.float32),
            pltpu.VMEM_SHARED((P,), jnp.float32),
            pltpu.SemaphoreType.DMA,
            pltpu.SemaphoreType.DMA,
        ],
    )(energies, ids)


def _tc_combine_body(p_ref, o_ref):
    o_ref[...] = p_ref[0] + p_ref[1]


@jax.jit
def _combine(partials_flat):
    return _tc_combine(partials_flat.reshape(NC, P))


@jax.jit
def _tc_combine(partials):
    return pl.pallas_call(
        _tc_combine_body,
        out_shape=jax.ShapeDtypeStruct((P,), jnp.float32),
    )(partials)


def kernel(atomic_energies, batch):
    ids = batch.astype(jnp.int32)
    partials = _sc_segment_sum(atomic_energies, ids)
    return _combine(partials)[:N_MOL]


# speedup vs baseline: 32.3949x; 1.0511x over previous
import functools

import jax
import jax.numpy as jnp
from jax import lax
from jax.experimental import pallas as pl
from jax.experimental.pallas import tpu as pltpu
from jax.experimental.pallas import tpu_sc as plsc

N_ATOMS = 1600000
N_MOL = 50000
NC = 2
NS = 16
L = 16
NW = NC * NS
CHUNK = N_ATOMS // NW
BLK = 5000
NBLK = CHUNK // BLK
P = 50176
PS = P // NS


def _sc_segment_sum_body(e_hbm, i_hbm, out_hbm,
                         e_v0, i_v0, e_v1, i_v1, z_v, acc_sh, sem0, sem1):
    c = lax.axis_index("c")
    s = lax.axis_index("s")
    wid = c * NS + s
    base = wid * CHUNK

    def start_load(b, e_v, i_v, sem):
        off = base + b * BLK
        pltpu.make_async_copy(e_hbm.at[pl.ds(off, BLK)], e_v, sem).start()
        pltpu.make_async_copy(i_hbm.at[pl.ds(off, BLK)], i_v, sem).start()

    def wait_load(e_v, i_v, sem):
        pltpu.make_async_copy(e_hbm.at[pl.ds(base, BLK)], e_v, sem).wait()
        pltpu.make_async_copy(i_hbm.at[pl.ds(base, BLK)], i_v, sem).wait()

    start_load(0, e_v0, i_v0, sem0)

    @pl.loop(0, PS, step=L)
    def _zero(j):
        z_v[pl.ds(j, L)] = jnp.zeros((L,), jnp.float32)

    pltpu.sync_copy(z_v, acc_sh.at[pl.ds(s * PS, PS)])
    plsc.subcore_barrier()

    @pl.loop(0, NBLK, step=2)
    def _block(b):
        wait_load(e_v0, i_v0, sem0)
        start_load(b + 1, e_v1, i_v1, sem1)
        pltpu.sync_copy(e_v0, acc_sh.at[i_v0], add=True)
        wait_load(e_v1, i_v1, sem1)

        @pl.when(b + 2 < NBLK)
        def _():
            start_load(b + 2, e_v0, i_v0, sem0)

        pltpu.sync_copy(e_v1, acc_sh.at[i_v1], add=True)

    plsc.subcore_barrier()
    pltpu.sync_copy(acc_sh.at[pl.ds(s * PS, PS)], z_v)
    pltpu.sync_copy(z_v, out_hbm.at[pl.ds(c * P + s * PS, PS)])


def _sc_segment_sum(energies, ids):
    mesh = plsc.VectorSubcoreMesh(core_axis_name="c", subcore_axis_name="s")
    return pl.kernel(
        _sc_segment_sum_body,
        out_type=jax.ShapeDtypeStruct((NC * P,), jnp.float32),
        mesh=mesh,
        scratch_types=[
            pltpu.VMEM((BLK,), jnp.float32),
            pltpu.VMEM((BLK,), jnp.int32),
            pltpu.VMEM((BLK,), jnp.float32),
            pltpu.VMEM((BLK,), jnp.int32),
            pltpu.VMEM((PS,), jnp.float32),
            pltpu.VMEM_SHARED((P,), jnp.float32),
            pltpu.SemaphoreType.DMA,
            pltpu.SemaphoreType.DMA,
        ],
    )(energies, ids)


def _tc_combine_body(p_ref, o_ref):
    o_ref[...] = p_ref[pl.ds(0, N_MOL)] + p_ref[pl.ds(P, N_MOL)]


def _tc_combine(partials_flat):
    return pl.pallas_call(
        _tc_combine_body,
        out_shape=jax.ShapeDtypeStruct((N_MOL,), jnp.float32),
    )(partials_flat)


@jax.jit
def _run(atomic_energies, ids):
    partials = _sc_segment_sum(atomic_energies, ids)
    return _tc_combine(partials)


def kernel(atomic_energies, batch):
    return _run(atomic_energies, batch.astype(jnp.int32))


# speedup vs baseline: 34.5697x; 1.0671x over previous
import functools

import jax
import jax.numpy as jnp
from jax import lax
from jax.experimental import pallas as pl
from jax.experimental.pallas import tpu as pltpu
from jax.experimental.pallas import tpu_sc as plsc

N_ATOMS = 1600000
N_MOL = 50000
NC = 2
NS = 16
L = 16
NW = NC * NS
CHUNK = N_ATOMS // NW
BLK = 5000
NBLK = CHUNK // BLK
P = 50176
PS = P // NS


def _sc_segment_sum_body(e_hbm, i_hbm, out_hbm,
                         e_v0, i_v0, e_v1, i_v1, z_v, acc_sh,
                         sem0, sem1, ssem0, ssem1):
    c = lax.axis_index("c")
    s = lax.axis_index("s")
    wid = c * NS + s
    base = wid * CHUNK

    def start_load(b, e_v, i_v, sem):
        off = base + b * BLK
        pltpu.make_async_copy(e_hbm.at[pl.ds(off, BLK)], e_v, sem).start()
        pltpu.make_async_copy(i_hbm.at[pl.ds(off, BLK)], i_v, sem).start()

    def wait_load(e_v, i_v, sem):
        pltpu.make_async_copy(e_hbm.at[pl.ds(base, BLK)], e_v, sem).wait()
        pltpu.make_async_copy(i_hbm.at[pl.ds(base, BLK)], i_v, sem).wait()

    def start_scatter(e_v, i_v, sem):
        pltpu.async_copy(e_v, acc_sh.at[i_v], sem, add=True)

    def wait_scatter(e_v, i_v, sem):
        pltpu.make_async_copy(e_v, acc_sh.at[i_v], sem).wait()

    start_load(0, e_v0, i_v0, sem0)
    start_load(1, e_v1, i_v1, sem1)

    @pl.loop(0, PS, step=L)
    def _zero(j):
        z_v[pl.ds(j, L)] = jnp.zeros((L,), jnp.float32)

    pltpu.sync_copy(z_v, acc_sh.at[pl.ds(s * PS, PS)])
    plsc.subcore_barrier()

    @pl.loop(0, NBLK, step=2)
    def _block(b):
        wait_load(e_v0, i_v0, sem0)
        start_scatter(e_v0, i_v0, ssem0)
        wait_load(e_v1, i_v1, sem1)
        start_scatter(e_v1, i_v1, ssem1)
        wait_scatter(e_v0, i_v0, ssem0)

        @pl.when(b + 2 < NBLK)
        def _():
            start_load(b + 2, e_v0, i_v0, sem0)

        wait_scatter(e_v1, i_v1, ssem1)

        @pl.when(b + 3 < NBLK)
        def _():
            start_load(b + 3, e_v1, i_v1, sem1)

    plsc.subcore_barrier()
    pltpu.sync_copy(acc_sh.at[pl.ds(s * PS, PS)], z_v)
    pltpu.sync_copy(z_v, out_hbm.at[pl.ds(c * P + s * PS, PS)])


def _sc_segment_sum(energies, ids):
    mesh = plsc.VectorSubcoreMesh(core_axis_name="c", subcore_axis_name="s")
    return pl.kernel(
        _sc_segment_sum_body,
        out_type=jax.ShapeDtypeStruct((NC * P,), jnp.float32),
        mesh=mesh,
        scratch_types=[
            pltpu.VMEM((BLK,), jnp.float32),
            pltpu.VMEM((BLK,), jnp.int32),
            pltpu.VMEM((BLK,), jnp.float32),
            pltpu.VMEM((BLK,), jnp.int32),
            pltpu.VMEM((PS,), jnp.float32),
            pltpu.VMEM_SHARED((P,), jnp.float32),
            pltpu.SemaphoreType.DMA,
            pltpu.SemaphoreType.DMA,
            pltpu.SemaphoreType.DMA,
            pltpu.SemaphoreType.DMA,
        ],
    )(energies, ids)


def _tc_combine_body(p_ref, o_ref):
    o_ref[...] = p_ref[pl.ds(0, N_MOL)] + p_ref[pl.ds(P, N_MOL)]


def _tc_combine(partials_flat):
    return pl.pallas_call(
        _tc_combine_body,
        out_shape=jax.ShapeDtypeStruct((N_MOL,), jnp.float32),
    )(partials_flat)


@jax.jit
def _run(atomic_energies, ids):
    partials = _sc_segment_sum(atomic_energies, ids)
    return _tc_combine(partials)


def kernel(atomic_energies, batch):
    return _run(atomic_energies, batch.astype(jnp.int32))
